# baseline (speedup 1.0000x reference)
import functools

import jax
import jax.numpy as jnp
from jax import lax
from jax.experimental import pallas as pl
from jax.experimental.pallas import tpu as pltpu
from jax.experimental.pallas import tpu_sc as plsc

_B = 16384
_K = 50
_D = 16
_NC, _NS = 2, 16
_NW = _NC * _NS
_RPW = _B // _NW
_CH = 32
_NCHUNK = _RPW // _CH
_CKI = _CH * _K
_REGS = 1e-05


def _sc_body(user_h, pos_h, negs_h, uemb_h, iemb_h, diff_h, reg_h,
             uidx_v, pidx_v, urows_v, prows_v,
             nidx0_v, nidx1_v, nrows0_v, nrows1_v,
             diff_v, reg_v, semu, semp, sem0, sem1):
    wid = lax.axis_index("s") * _NC + lax.axis_index("c")
    base = wid * _RPW
    nbase = base * _K

    pltpu.sync_copy(user_h.at[pl.ds(base, _RPW)], uidx_v)
    pltpu.sync_copy(pos_h.at[pl.ds(base, _RPW)], pidx_v)
    pltpu.async_copy(uemb_h.at[uidx_v], urows_v, semu)
    pltpu.async_copy(iemb_h.at[pidx_v], prows_v, semp)

    pltpu.sync_copy(negs_h.at[pl.ds(nbase, _CKI)], nidx0_v)
    pltpu.async_copy(iemb_h.at[nidx0_v], nrows0_v, sem0)
    pltpu.sync_copy(negs_h.at[pl.ds(nbase + _CKI, _CKI)], nidx1_v)
    pltpu.async_copy(iemb_h.at[nidx1_v], nrows1_v, sem1)

    pltpu.make_async_copy(uemb_h.at[uidx_v], urows_v, semu).wait()
    pltpu.make_async_copy(iemb_h.at[pidx_v], prows_v, semp).wait()

    lanes = lax.iota(jnp.int32, 16)
    dcol = [jnp.full((16,), d, jnp.int32) for d in range(_D)]

    def compute_chunk(c, nrows_v):
        for g in range(_CH // 16):
            row0 = c * _CH + g * 16
            rows_vec = row0 + lanes
            u_t = [plsc.load_gather(urows_v, [rows_vec, dcol[d]])
                   for d in range(_D)]
            nvec = (g * 16 + lanes) * _K

            def kbody(k, carry):
                best, bestn = carry
                i0 = nvec + k
                acc = jnp.zeros((16,), jnp.float32)
                nrm = jnp.zeros((16,), jnp.float32)
                for d in range(_D):
                    v = plsc.load_gather(nrows_v, [i0, dcol[d]])
                    acc = acc + v * u_t[d]
                    nrm = nrm + v * v
                upd = acc > best
                best = jnp.where(upd, acc, best)
                bestn = jnp.where(upd, nrm, bestn)
                return best, bestn

            best, bestn = lax.fori_loop(
                0, _K, kbody,
                (jnp.full((16,), -jnp.inf, jnp.float32),
                 jnp.zeros((16,), jnp.float32)))

            pdot = jnp.zeros((16,), jnp.float32)
            pnrm = jnp.zeros((16,), jnp.float32)
            unrm = jnp.zeros((16,), jnp.float32)
            for d in range(_D):
                pv = plsc.load_gather(prows_v, [rows_vec, dcol[d]])
                pdot = pdot + pv * u_t[d]
                pnrm = pnrm + pv * pv
                unrm = unrm + u_t[d] * u_t[d]
            diff_v[pl.ds(row0, 16)] = pdot - best
            reg_v[pl.ds(row0, 16)] = unrm + pnrm + bestn

    def outer(i, carry):
        for b, (nidx, nrows, sem) in enumerate(
                ((nidx0_v, nrows0_v, sem0), (nidx1_v, nrows1_v, sem1))):
            c = i * 2 + b
            pltpu.make_async_copy(iemb_h.at[nidx], nrows, sem).wait()
            compute_chunk(c, nrows)
            nxt = c + 2

            @pl.when(nxt < _NCHUNK)
            def _():
                pltpu.sync_copy(negs_h.at[pl.ds(nbase + nxt * _CKI, _CKI)],
                                nidx)
                pltpu.async_copy(iemb_h.at[nidx], nrows, sem)
        return carry

    lax.fori_loop(0, _NCHUNK // 2, outer, 0)

    pltpu.sync_copy(diff_v, diff_h.at[pl.ds(base, _RPW)])
    pltpu.sync_copy(reg_v, reg_h.at[pl.ds(base, _RPW)])


_sc_fn = pl.kernel(
    _sc_body,
    out_type=[jax.ShapeDtypeStruct((_B,), jnp.float32),
              jax.ShapeDtypeStruct((_B,), jnp.float32)],
    mesh=plsc.VectorSubcoreMesh(core_axis_name="c", subcore_axis_name="s"),
    scratch_types=[
        pltpu.VMEM((_RPW,), jnp.int32),
        pltpu.VMEM((_RPW,), jnp.int32),
        pltpu.VMEM((_RPW, _D), jnp.float32),
        pltpu.VMEM((_RPW, _D), jnp.float32),
        pltpu.VMEM((_CKI,), jnp.int32),
        pltpu.VMEM((_CKI,), jnp.int32),
        pltpu.VMEM((_CKI, _D), jnp.float32),
        pltpu.VMEM((_CKI, _D), jnp.float32),
        pltpu.VMEM((_RPW,), jnp.float32),
        pltpu.VMEM((_RPW,), jnp.float32),
        pltpu.SemaphoreType.DMA,
        pltpu.SemaphoreType.DMA,
        pltpu.SemaphoreType.DMA,
        pltpu.SemaphoreType.DMA,
    ],
    compiler_params=pltpu.CompilerParams(use_tc_tiling_on_sc=False,
                                         needs_layout_passes=False),
)


def _tc_body(diff_ref, reg_ref, loss_ref, regl_ref):
    x = diff_ref[...]
    sp = jnp.maximum(-x, 0.0) + jnp.log(1.0 + jnp.exp(-jnp.abs(x)))
    loss_ref[...] = jnp.reshape(jnp.sum(sp) * (1.0 / _B), (1, 1))
    regl_ref[...] = jnp.reshape(
        jnp.sum(reg_ref[...]) * jnp.float32(_REGS * 0.5 / _B), (1, 1))


_tc_fn = pl.pallas_call(
    _tc_body,
    out_shape=(jax.ShapeDtypeStruct((1, 1), jnp.float32),
               jax.ShapeDtypeStruct((1, 1), jnp.float32)),
)


def kernel(user, pos, negs, user_embedding, item_embedding):
    user = user.astype(jnp.int32)
    pos = pos.astype(jnp.int32)
    negs_flat = negs.astype(jnp.int32).reshape(-1)
    diff, reg = _sc_fn(user, pos, negs_flat, user_embedding, item_embedding)
    loss, regl = _tc_fn(diff.reshape(128, 128), reg.reshape(128, 128))
    return (loss[0, 0], regl[0, 0])
